# initial kernel scaffold (unmeasured)
import jax
import jax.numpy as jnp
from jax import lax
from jax.experimental import pallas as pl
from jax.experimental.pallas import tpu as pltpu

N_DEV = 4


def kernel(x, w_mat, scale_x, scale_w):
    m_per, k = x.shape
    _, n_per = w_mat.shape
    half = m_per // 2
    m_tot = N_DEV * m_per

    def body(x_ref, w_ref, sx_ref, sw_ref, out_ref,
             x8_ref, rl_ref, rr_ref, opp_ref, wb_ref,
             send_sems, recv_sems):
        my = lax.axis_index("i")
        left = lax.rem(my + N_DEV - 1, N_DEV)
        right = lax.rem(my + 1, N_DEV)

        barrier_sem = pltpu.get_barrier_semaphore()
        for nbr in (left, right):
            pl.semaphore_signal(
                barrier_sem, inc=1,
                device_id=(nbr,), device_id_type=pl.DeviceIdType.MESH,
            )
        pl.semaphore_wait(barrier_sem, 2)

        x8_ref[...] = x_ref[...].astype(jnp.float8_e5m2)
        wb_ref[...] = w_ref[...].astype(jnp.bfloat16)
        scale = sx_ref[0] * sw_ref[0]

        def gemm_to(origin, chunk_bf16):
            acc = lax.dot_general(
                chunk_bf16, wb_ref[...],
                dimension_numbers=(((1,), (0,)), ((), ())),
                preferred_element_type=jnp.float32,
            )
            out_ref[pl.ds(origin * m_per, m_per), :] = acc * scale

        sr1 = pltpu.make_async_remote_copy(
            src_ref=x8_ref, dst_ref=rl_ref,
            send_sem=send_sems.at[0], recv_sem=recv_sems.at[0],
            device_id=(right,), device_id_type=pl.DeviceIdType.MESH,
        )
        sl1 = pltpu.make_async_remote_copy(
            src_ref=x8_ref, dst_ref=rr_ref,
            send_sem=send_sems.at[1], recv_sem=recv_sems.at[1],
            device_id=(left,), device_id_type=pl.DeviceIdType.MESH,
        )
        sr1.start()
        sl1.start()

        gemm_to(my, x_ref[...].astype(jnp.bfloat16))

        sr1.wait_recv()
        sr2 = pltpu.make_async_remote_copy(
            src_ref=rl_ref.at[pl.ds(0, half)],
            dst_ref=opp_ref.at[pl.ds(0, half)],
            send_sem=send_sems.at[2], recv_sem=recv_sems.at[2],
            device_id=(right,), device_id_type=pl.DeviceIdType.MESH,
        )
        sr2.start()
        gemm_to(left, rl_ref[...].astype(jnp.bfloat16))

        sl1.wait_recv()
        sl2 = pltpu.make_async_remote_copy(
            src_ref=rr_ref.at[pl.ds(half, half)],
            dst_ref=opp_ref.at[pl.ds(half, half)],
            send_sem=send_sems.at[3], recv_sem=recv_sems.at[3],
            device_id=(left,), device_id_type=pl.DeviceIdType.MESH,
        )
        sl2.start()
        gemm_to(right, rr_ref[...].astype(jnp.bfloat16))

        sr2.wait_recv()
        sl2.wait_recv()
        gemm_to(lax.rem(my + 2, N_DEV), opp_ref[...].astype(jnp.bfloat16))

        sr1.wait_send()
        sl1.wait_send()
        sr2.wait_send()
        sl2.wait_send()

    return pl.pallas_call(
        body,
        out_shape=jax.ShapeDtypeStruct((m_tot, n_per), jnp.float32),
        in_specs=[
            pl.BlockSpec(memory_space=pltpu.VMEM),
            pl.BlockSpec(memory_space=pltpu.VMEM),
            pl.BlockSpec(memory_space=pltpu.SMEM),
            pl.BlockSpec(memory_space=pltpu.SMEM),
        ],
        out_specs=pl.BlockSpec(memory_space=pltpu.VMEM),
        scratch_shapes=[
            pltpu.VMEM((m_per, k), jnp.float8_e5m2),
            pltpu.VMEM((m_per, k), jnp.float8_e5m2),
            pltpu.VMEM((m_per, k), jnp.float8_e5m2),
            pltpu.VMEM((m_per, k), jnp.float8_e5m2),
            pltpu.VMEM((k, n_per), jnp.bfloat16),
            pltpu.SemaphoreType.DMA((4,)),
            pltpu.SemaphoreType.DMA((4,)),
        ],
        compiler_params=pltpu.CompilerParams(collective_id=0),
    )(x, w_mat, scale_x, scale_w)


# baseline (device time: 97963 ns/iter reference)
import jax
import jax.numpy as jnp
from jax import lax
from jax.experimental import pallas as pl
from jax.experimental.pallas import tpu as pltpu

N_DEV = 4


def kernel(x, w_mat, scale_x, scale_w):
    m_per, k = x.shape
    _, n_per = w_mat.shape
    half = m_per // 2
    m_tot = N_DEV * m_per

    def body(x_ref, w_ref, sx_ref, sw_ref, out_ref,
             x8_ref, rl_ref, rr_ref, opp_ref, wb_ref,
             send_sems, recv_sems):
        my = lax.axis_index("i")
        left = lax.rem(my + N_DEV - 1, N_DEV)
        right = lax.rem(my + 1, N_DEV)

        barrier_sem = pltpu.get_barrier_semaphore()
        for nbr in (left, right):
            pl.semaphore_signal(
                barrier_sem, inc=1,
                device_id=(nbr,), device_id_type=pl.DeviceIdType.MESH,
            )
        pl.semaphore_wait(barrier_sem, 2)

        x8_ref[...] = x_ref[...].astype(jnp.float8_e5m2)
        wb_ref[...] = w_ref[...].astype(jnp.float8_e5m2)
        scale = sx_ref[0] * sw_ref[0]

        def gemm_to(origin, chunk_fp8):
            acc = lax.dot_general(
                chunk_fp8, wb_ref[...],
                dimension_numbers=(((1,), (0,)), ((), ())),
                preferred_element_type=jnp.float32,
            )
            out_ref[pl.ds(origin * m_per, m_per), :] = acc * scale

        sr1 = pltpu.make_async_remote_copy(
            src_ref=x8_ref, dst_ref=rl_ref,
            send_sem=send_sems.at[0], recv_sem=recv_sems.at[0],
            device_id=(right,), device_id_type=pl.DeviceIdType.MESH,
        )
        sl1 = pltpu.make_async_remote_copy(
            src_ref=x8_ref, dst_ref=rr_ref,
            send_sem=send_sems.at[1], recv_sem=recv_sems.at[1],
            device_id=(left,), device_id_type=pl.DeviceIdType.MESH,
        )
        sr1.start()
        sl1.start()

        gemm_to(my, x8_ref[...])

        sr1.wait_recv()
        sr2 = pltpu.make_async_remote_copy(
            src_ref=rl_ref.at[pl.ds(0, half)],
            dst_ref=opp_ref.at[pl.ds(0, half)],
            send_sem=send_sems.at[2], recv_sem=recv_sems.at[2],
            device_id=(right,), device_id_type=pl.DeviceIdType.MESH,
        )
        sr2.start()
        gemm_to(left, rl_ref[...])

        sl1.wait_recv()
        sl2 = pltpu.make_async_remote_copy(
            src_ref=rr_ref.at[pl.ds(half, half)],
            dst_ref=opp_ref.at[pl.ds(half, half)],
            send_sem=send_sems.at[3], recv_sem=recv_sems.at[3],
            device_id=(left,), device_id_type=pl.DeviceIdType.MESH,
        )
        sl2.start()
        gemm_to(right, rr_ref[...])

        sr2.wait_recv()
        sl2.wait_recv()
        gemm_to(lax.rem(my + 2, N_DEV), opp_ref[...])

        sr1.wait_send()
        sl1.wait_send()
        sr2.wait_send()
        sl2.wait_send()

    return pl.pallas_call(
        body,
        out_shape=jax.ShapeDtypeStruct((m_tot, n_per), jnp.float32),
        in_specs=[
            pl.BlockSpec(memory_space=pltpu.VMEM),
            pl.BlockSpec(memory_space=pltpu.VMEM),
            pl.BlockSpec(memory_space=pltpu.SMEM),
            pl.BlockSpec(memory_space=pltpu.SMEM),
        ],
        out_specs=pl.BlockSpec(memory_space=pltpu.VMEM),
        scratch_shapes=[
            pltpu.VMEM((m_per, k), jnp.float8_e5m2),
            pltpu.VMEM((m_per, k), jnp.float8_e5m2),
            pltpu.VMEM((m_per, k), jnp.float8_e5m2),
            pltpu.VMEM((m_per, k), jnp.float8_e5m2),
            pltpu.VMEM((k, n_per), jnp.float8_e5m2),
            pltpu.SemaphoreType.DMA((4,)),
            pltpu.SemaphoreType.DMA((4,)),
        ],
        compiler_params=pltpu.CompilerParams(
            collective_id=0,
            vmem_limit_bytes=100 * 1024 * 1024,
        ),
    )(x, w_mat, scale_x, scale_w)


# device time: 82128 ns/iter; 1.1928x vs baseline; 1.1928x over previous
import jax
import jax.numpy as jnp
from jax import lax
from jax.experimental import pallas as pl
from jax.experimental.pallas import tpu as pltpu

N_DEV = 4


def kernel(x, w_mat, scale_x, scale_w):
    m_per, k = x.shape
    _, n_per = w_mat.shape
    kh = k // 2
    m_tot = N_DEV * m_per

    def body(x_ref, w_ref, sx_ref, sw_ref, out_ref,
             x8_ref, w8_ref, wl_ref, wr_ref, wo_ref,
             bl_ref, br_ref, bo_ref, rbl_ref, rbr_ref, rbo_ref,
             send_sems, recv_sems):
        my = lax.axis_index("i")
        left = lax.rem(my + N_DEV - 1, N_DEV)
        right = lax.rem(my + 1, N_DEV)
        opp = lax.rem(my + 2, N_DEV)

        barrier_sem = pltpu.get_barrier_semaphore()
        for nbr in (left, right):
            pl.semaphore_signal(
                barrier_sem, inc=1,
                device_id=(nbr,), device_id_type=pl.DeviceIdType.MESH,
            )
        pl.semaphore_wait(barrier_sem, 2)

        def copy(src, dst, i, dev):
            return pltpu.make_async_remote_copy(
                src_ref=src, dst_ref=dst,
                send_sem=send_sems.at[i], recv_sem=recv_sems.at[i],
                device_id=(dev,), device_id_type=pl.DeviceIdType.MESH,
            )

        w8_ref[...] = w_ref[...].astype(jnp.float8_e5m2)
        wrA = copy(w8_ref.at[pl.ds(0, kh)], wl_ref.at[pl.ds(0, kh)], 0, right)
        wrB = copy(w8_ref.at[pl.ds(kh, kh)], wl_ref.at[pl.ds(kh, kh)], 1, right)
        wlA = copy(w8_ref.at[pl.ds(0, kh)], wr_ref.at[pl.ds(0, kh)], 2, left)
        wlB = copy(w8_ref.at[pl.ds(kh, kh)], wr_ref.at[pl.ds(kh, kh)], 3, left)
        wrA.start()
        wrB.start()
        wlA.start()
        wlB.start()

        x8_ref[...] = x_ref[...].astype(jnp.float8_e5m2)
        scale = sx_ref[0] * sw_ref[0]

        def gemm(w_block_ref):
            return lax.dot_general(
                x8_ref[...], w_block_ref[...],
                dimension_numbers=(((1,), (0,)), ((), ())),
                preferred_element_type=jnp.float32,
            ) * scale

        out_ref[pl.ds(my * m_per, m_per), :] = gemm(w8_ref)

        wrA.wait_recv()
        fr = copy(wl_ref.at[pl.ds(0, kh)], wo_ref.at[pl.ds(0, kh)], 4, right)
        fr.start()
        wlB.wait_recv()
        fl = copy(wr_ref.at[pl.ds(kh, kh)], wo_ref.at[pl.ds(kh, kh)], 5, left)
        fl.start()

        wrB.wait_recv()
        bl_ref[...] = gemm(wl_ref).astype(jnp.bfloat16)
        sbl = copy(bl_ref, rbr_ref, 6, left)
        sbl.start()

        wlA.wait_recv()
        br_ref[...] = gemm(wr_ref).astype(jnp.bfloat16)
        sbr = copy(br_ref, rbl_ref, 7, right)
        sbr.start()

        fr.wait_recv()
        fl.wait_recv()
        bo_ref[...] = gemm(wo_ref).astype(jnp.bfloat16)
        sbo = copy(bo_ref, rbo_ref, 8, opp)
        sbo.start()

        sbr.wait_recv()
        out_ref[pl.ds(left * m_per, m_per), :] = rbl_ref[...].astype(jnp.float32)
        sbl.wait_recv()
        out_ref[pl.ds(right * m_per, m_per), :] = rbr_ref[...].astype(jnp.float32)
        sbo.wait_recv()
        out_ref[pl.ds(opp * m_per, m_per), :] = rbo_ref[...].astype(jnp.float32)

        for d in (wrA, wrB, wlA, wlB, fr, fl, sbl, sbr, sbo):
            d.wait_send()

    return pl.pallas_call(
        body,
        out_shape=jax.ShapeDtypeStruct((m_tot, n_per), jnp.float32),
        in_specs=[
            pl.BlockSpec(memory_space=pltpu.VMEM),
            pl.BlockSpec(memory_space=pltpu.VMEM),
            pl.BlockSpec(memory_space=pltpu.SMEM),
            pl.BlockSpec(memory_space=pltpu.SMEM),
        ],
        out_specs=pl.BlockSpec(memory_space=pltpu.VMEM),
        scratch_shapes=[
            pltpu.VMEM((m_per, k), jnp.float8_e5m2),
            pltpu.VMEM((k, n_per), jnp.float8_e5m2),
            pltpu.VMEM((k, n_per), jnp.float8_e5m2),
            pltpu.VMEM((k, n_per), jnp.float8_e5m2),
            pltpu.VMEM((k, n_per), jnp.float8_e5m2),
            pltpu.VMEM((m_per, n_per), jnp.bfloat16),
            pltpu.VMEM((m_per, n_per), jnp.bfloat16),
            pltpu.VMEM((m_per, n_per), jnp.bfloat16),
            pltpu.VMEM((m_per, n_per), jnp.bfloat16),
            pltpu.VMEM((m_per, n_per), jnp.bfloat16),
            pltpu.VMEM((m_per, n_per), jnp.bfloat16),
            pltpu.SemaphoreType.DMA((9,)),
            pltpu.SemaphoreType.DMA((9,)),
        ],
        compiler_params=pltpu.CompilerParams(
            collective_id=0,
            vmem_limit_bytes=100 * 1024 * 1024,
        ),
    )(x, w_mat, scale_x, scale_w)


# device time: 81888 ns/iter; 1.1963x vs baseline; 1.0029x over previous
import jax
import jax.numpy as jnp
from jax import lax
from jax.experimental import pallas as pl
from jax.experimental.pallas import tpu as pltpu

N_DEV = 4


def kernel(x, w_mat, scale_x, scale_w):
    m_per, k = x.shape
    _, n_per = w_mat.shape
    kh = k // 2
    m_tot = N_DEV * m_per

    def body(x_ref, w_ref, sx_ref, sw_ref, out_ref,
             x8_ref, w8_ref, wl_ref, wr_ref, wo_ref,
             bl_ref, br_ref, bo_ref, rbl_ref, rbr_ref, rbo_ref,
             send_sems, recv_sems):
        my = lax.axis_index("i")
        left = lax.rem(my + N_DEV - 1, N_DEV)
        right = lax.rem(my + 1, N_DEV)
        opp = lax.rem(my + 2, N_DEV)

        barrier_sem = pltpu.get_barrier_semaphore()
        for nbr in (left, right):
            pl.semaphore_signal(
                barrier_sem, inc=1,
                device_id=(nbr,), device_id_type=pl.DeviceIdType.MESH,
            )
        pl.semaphore_wait(barrier_sem, 2)

        def copy(src, dst, i, dev):
            return pltpu.make_async_remote_copy(
                src_ref=src, dst_ref=dst,
                send_sem=send_sems.at[i], recv_sem=recv_sems.at[i],
                device_id=(dev,), device_id_type=pl.DeviceIdType.MESH,
            )

        w8_ref[...] = w_ref[...].astype(jnp.float8_e5m2)
        wrA = copy(w8_ref.at[pl.ds(0, kh)], wl_ref.at[pl.ds(0, kh)], 0, right)
        wrB = copy(w8_ref.at[pl.ds(kh, kh)], wl_ref.at[pl.ds(kh, kh)], 1, right)
        wlA = copy(w8_ref.at[pl.ds(0, kh)], wr_ref.at[pl.ds(0, kh)], 2, left)
        wlB = copy(w8_ref.at[pl.ds(kh, kh)], wr_ref.at[pl.ds(kh, kh)], 3, left)
        wrA.start()
        wrB.start()
        wlA.start()
        wlB.start()

        x8_ref[...] = x_ref[...].astype(jnp.float8_e5m2)
        scale = sx_ref[0] * sw_ref[0]

        mh = m_per // 2

        def gemm(w_block_ref, r0=0, rows=m_per):
            return lax.dot_general(
                x8_ref[pl.ds(r0, rows), :], w_block_ref[...],
                dimension_numbers=(((1,), (0,)), ((), ())),
                preferred_element_type=jnp.float32,
            ) * scale

        out_ref[pl.ds(my * m_per, m_per), :] = gemm(w8_ref)

        wrA.wait_recv()
        fr = copy(wl_ref.at[pl.ds(0, kh)], wo_ref.at[pl.ds(0, kh)], 4, right)
        fr.start()
        wlB.wait_recv()
        fl = copy(wr_ref.at[pl.ds(kh, kh)], wo_ref.at[pl.ds(kh, kh)], 5, left)
        fl.start()

        def block_halves(w_src, b_ref, r_ref, sem0, dev):
            descs = []
            for i, r0 in enumerate((0, mh)):
                b_ref[pl.ds(r0, mh), :] = gemm(
                    w_src, r0, mh).astype(jnp.bfloat16)
                d = copy(b_ref.at[pl.ds(r0, mh)], r_ref.at[pl.ds(r0, mh)],
                         sem0 + i, dev)
                d.start()
                descs.append(d)
            return descs

        wrB.wait_recv()
        sbl = block_halves(wl_ref, bl_ref, rbr_ref, 6, left)

        wlA.wait_recv()
        sbr = block_halves(wr_ref, br_ref, rbl_ref, 8, right)

        fr.wait_recv()
        fl.wait_recv()
        sbo = block_halves(wo_ref, bo_ref, rbo_ref, 10, opp)

        for d in sbr:
            d.wait_recv()
        out_ref[pl.ds(left * m_per, m_per), :] = rbl_ref[...].astype(jnp.float32)
        for d in sbl:
            d.wait_recv()
        out_ref[pl.ds(right * m_per, m_per), :] = rbr_ref[...].astype(jnp.float32)
        for d in sbo:
            d.wait_recv()
        out_ref[pl.ds(opp * m_per, m_per), :] = rbo_ref[...].astype(jnp.float32)

        for d in (wrA, wrB, wlA, wlB, fr, fl, *sbl, *sbr, *sbo):
            d.wait_send()

    return pl.pallas_call(
        body,
        out_shape=jax.ShapeDtypeStruct((m_tot, n_per), jnp.float32),
        in_specs=[
            pl.BlockSpec(memory_space=pltpu.VMEM),
            pl.BlockSpec(memory_space=pltpu.VMEM),
            pl.BlockSpec(memory_space=pltpu.SMEM),
            pl.BlockSpec(memory_space=pltpu.SMEM),
        ],
        out_specs=pl.BlockSpec(memory_space=pltpu.VMEM),
        scratch_shapes=[
            pltpu.VMEM((m_per, k), jnp.float8_e5m2),
            pltpu.VMEM((k, n_per), jnp.float8_e5m2),
            pltpu.VMEM((k, n_per), jnp.float8_e5m2),
            pltpu.VMEM((k, n_per), jnp.float8_e5m2),
            pltpu.VMEM((k, n_per), jnp.float8_e5m2),
            pltpu.VMEM((m_per, n_per), jnp.bfloat16),
            pltpu.VMEM((m_per, n_per), jnp.bfloat16),
            pltpu.VMEM((m_per, n_per), jnp.bfloat16),
            pltpu.VMEM((m_per, n_per), jnp.bfloat16),
            pltpu.VMEM((m_per, n_per), jnp.bfloat16),
            pltpu.VMEM((m_per, n_per), jnp.bfloat16),
            pltpu.SemaphoreType.DMA((12,)),
            pltpu.SemaphoreType.DMA((12,)),
        ],
        compiler_params=pltpu.CompilerParams(
            collective_id=0,
            vmem_limit_bytes=100 * 1024 * 1024,
        ),
    )(x, w_mat, scale_x, scale_w)


# device time: 80005 ns/iter; 1.2245x vs baseline; 1.0235x over previous
import jax
import jax.numpy as jnp
from jax import lax
from jax.experimental import pallas as pl
from jax.experimental.pallas import tpu as pltpu

N_DEV = 4


def kernel(x, w_mat, scale_x, scale_w):
    m_per, k = x.shape
    _, n_per = w_mat.shape
    kh = k // 2
    kq = k // 4
    m_tot = N_DEV * m_per

    def body(x_ref, w_ref, sx_ref, sw_ref, out_ref,
             x8_ref, w8_ref, wl_ref, wr_ref, wo_ref,
             bl_ref, br_ref, bo_ref, rbl_ref, rbr_ref, rbo_ref,
             acc_ref, send_sems, recv_sems):
        my = lax.axis_index("i")
        left = lax.rem(my + N_DEV - 1, N_DEV)
        right = lax.rem(my + 1, N_DEV)
        opp = lax.rem(my + 2, N_DEV)

        barrier_sem = pltpu.get_barrier_semaphore()
        for nbr in (left, right):
            pl.semaphore_signal(
                barrier_sem, inc=1,
                device_id=(nbr,), device_id_type=pl.DeviceIdType.MESH,
            )
        pl.semaphore_wait(barrier_sem, 2)

        def copy(src, dst, i, dev):
            return pltpu.make_async_remote_copy(
                src_ref=src, dst_ref=dst,
                send_sem=send_sems.at[i], recv_sem=recv_sems.at[i],
                device_id=(dev,), device_id_type=pl.DeviceIdType.MESH,
            )

        w8_ref[...] = w_ref[...].astype(jnp.float8_e5m2)
        wrA = copy(w8_ref.at[pl.ds(0, kh)], wl_ref.at[pl.ds(0, kh)], 0, right)
        wrB = copy(w8_ref.at[pl.ds(kh, kh)], wl_ref.at[pl.ds(kh, kh)], 1, right)
        wlB = copy(w8_ref.at[pl.ds(kh, kh)], wr_ref.at[pl.ds(kh, kh)], 2, left)
        wlA = copy(w8_ref.at[pl.ds(0, kh)], wr_ref.at[pl.ds(0, kh)], 3, left)
        wrA.start()
        wlB.start()
        wrB.start()
        wlA.start()

        x8_ref[...] = x_ref[...].astype(jnp.float8_e5m2)
        scale = sx_ref[0] * sw_ref[0]

        def dot(c0, cn, w_block_ref, r0, rn):
            return lax.dot_general(
                x8_ref[:, pl.ds(c0, cn)], w_block_ref[pl.ds(r0, rn), :],
                dimension_numbers=(((1,), (0,)), ((), ())),
                preferred_element_type=jnp.float32,
            )

        out_ref[pl.ds(my * m_per, m_per), :] = dot(0, k, w8_ref, 0, k) * scale

        wrA.wait_recv()
        fr1 = copy(wl_ref.at[pl.ds(0, kq)], wo_ref.at[pl.ds(0, kq)], 4, right)
        fr2 = copy(wl_ref.at[pl.ds(kq, kq)], wo_ref.at[pl.ds(kq, kq)], 5, right)
        fr1.start()
        fr2.start()
        wlB.wait_recv()
        fl1 = copy(wr_ref.at[pl.ds(kh, kq)], wo_ref.at[pl.ds(kh, kq)], 6, left)
        fl2 = copy(wr_ref.at[pl.ds(kh + kq, kq)], wo_ref.at[pl.ds(kh + kq, kq)], 7, left)
        fl1.start()
        fl2.start()

        wrB.wait_recv()
        bl_ref[...] = (dot(0, k, wl_ref, 0, k) * scale).astype(jnp.bfloat16)
        sbl = copy(bl_ref, rbr_ref, 8, left)
        sbl.start()
        wlA.wait_recv()
        br_ref[...] = (dot(0, k, wr_ref, 0, k) * scale).astype(jnp.bfloat16)
        sbr = copy(br_ref, rbl_ref, 9, right)
        sbr.start()

        fr1.wait_recv()
        fl1.wait_recv()
        acc_ref[...] = dot(0, kq, wo_ref, 0, kq) + dot(kh, kq, wo_ref, kh, kq)
        fr2.wait_recv()
        fl2.wait_recv()
        bo_ref[...] = (
            (acc_ref[...]
             + dot(kq, kq, wo_ref, kq, kq)
             + dot(kh + kq, kq, wo_ref, kh + kq, kq)) * scale
        ).astype(jnp.bfloat16)
        sbo = copy(bo_ref, rbo_ref, 10, opp)
        sbo.start()

        sbr.wait_recv()
        out_ref[pl.ds(left * m_per, m_per), :] = rbl_ref[...].astype(jnp.float32)
        sbl.wait_recv()
        out_ref[pl.ds(right * m_per, m_per), :] = rbr_ref[...].astype(jnp.float32)
        sbo.wait_recv()
        out_ref[pl.ds(opp * m_per, m_per), :] = rbo_ref[...].astype(jnp.float32)

        for d in (wrA, wrB, wlB, wlA, fr1, fr2, fl1, fl2, sbl, sbr, sbo):
            d.wait_send()

    return pl.pallas_call(
        body,
        out_shape=jax.ShapeDtypeStruct((m_tot, n_per), jnp.float32),
        in_specs=[
            pl.BlockSpec(memory_space=pltpu.VMEM),
            pl.BlockSpec(memory_space=pltpu.VMEM),
            pl.BlockSpec(memory_space=pltpu.SMEM),
            pl.BlockSpec(memory_space=pltpu.SMEM),
        ],
        out_specs=pl.BlockSpec(memory_space=pltpu.VMEM),
        scratch_shapes=[
            pltpu.VMEM((m_per, k), jnp.float8_e5m2),
            pltpu.VMEM((k, n_per), jnp.float8_e5m2),
            pltpu.VMEM((k, n_per), jnp.float8_e5m2),
            pltpu.VMEM((k, n_per), jnp.float8_e5m2),
            pltpu.VMEM((k, n_per), jnp.float8_e5m2),
            pltpu.VMEM((m_per, n_per), jnp.bfloat16),
            pltpu.VMEM((m_per, n_per), jnp.bfloat16),
            pltpu.VMEM((m_per, n_per), jnp.bfloat16),
            pltpu.VMEM((m_per, n_per), jnp.bfloat16),
            pltpu.VMEM((m_per, n_per), jnp.bfloat16),
            pltpu.VMEM((m_per, n_per), jnp.bfloat16),
            pltpu.VMEM((m_per, n_per), jnp.float32),
            pltpu.SemaphoreType.DMA((11,)),
            pltpu.SemaphoreType.DMA((11,)),
        ],
        compiler_params=pltpu.CompilerParams(
            collective_id=0,
            vmem_limit_bytes=100 * 1024 * 1024,
        ),
    )(x, w_mat, scale_x, scale_w)


# device time: 79920 ns/iter; 1.2258x vs baseline; 1.0011x over previous
import jax
import jax.numpy as jnp
from jax import lax
from jax.experimental import pallas as pl
from jax.experimental.pallas import tpu as pltpu

N_DEV = 4


def kernel(x, w_mat, scale_x, scale_w):
    m_per, k = x.shape
    _, n_per = w_mat.shape
    kh = k // 2
    kq = k // 4
    m_tot = N_DEV * m_per

    def body(x_ref, w_ref, sx_ref, sw_ref, out_ref,
             x8_ref, w8_ref, wl_ref, wr_ref, wo_ref,
             bl_ref, br_ref, bo_ref, rbl_ref, rbr_ref, rbo_ref,
             acc_ref, send_sems, recv_sems):
        my = lax.axis_index("i")
        left = lax.rem(my + N_DEV - 1, N_DEV)
        right = lax.rem(my + 1, N_DEV)
        opp = lax.rem(my + 2, N_DEV)

        barrier_sem = pltpu.get_barrier_semaphore()
        for nbr in (left, right):
            pl.semaphore_signal(
                barrier_sem, inc=1,
                device_id=(nbr,), device_id_type=pl.DeviceIdType.MESH,
            )
        pl.semaphore_wait(barrier_sem, 2)

        def copy(src, dst, i, dev):
            return pltpu.make_async_remote_copy(
                src_ref=src, dst_ref=dst,
                send_sem=send_sems.at[i], recv_sem=recv_sems.at[i],
                device_id=(dev,), device_id_type=pl.DeviceIdType.MESH,
            )

        w8_ref[...] = w_ref[...].astype(jnp.float8_e5m2)
        wrA = copy(w8_ref.at[pl.ds(0, kh)], wl_ref.at[pl.ds(0, kh)], 0, right)
        wrB = copy(w8_ref.at[pl.ds(kh, kh)], wl_ref.at[pl.ds(kh, kh)], 1, right)
        wlB = copy(w8_ref.at[pl.ds(kh, kh)], wr_ref.at[pl.ds(kh, kh)], 2, left)
        wlA = copy(w8_ref.at[pl.ds(0, kh)], wr_ref.at[pl.ds(0, kh)], 3, left)
        wrA.start()
        wlB.start()
        wrB.start()
        wlA.start()

        x8_ref[...] = x_ref[...].astype(jnp.float8_e5m2)
        scale = sx_ref[0] * sw_ref[0]

        def dot(c0, cn, w_block_ref, r0, rn, xr0=0, xrn=m_per):
            return lax.dot_general(
                x8_ref[pl.ds(xr0, xrn), pl.ds(c0, cn)],
                w_block_ref[pl.ds(r0, rn), :],
                dimension_numbers=(((1,), (0,)), ((), ())),
                preferred_element_type=jnp.float32,
            )

        out_ref[pl.ds(my * m_per, m_per), :] = dot(0, k, w8_ref, 0, k) * scale

        wrA.wait_recv()
        fr1 = copy(wl_ref.at[pl.ds(0, kq)], wo_ref.at[pl.ds(0, kq)], 4, right)
        fr2 = copy(wl_ref.at[pl.ds(kq, kq)], wo_ref.at[pl.ds(kq, kq)], 5, right)
        fr1.start()
        fr2.start()
        wlB.wait_recv()
        fl1 = copy(wr_ref.at[pl.ds(kh, kq)], wo_ref.at[pl.ds(kh, kq)], 6, left)
        fl2 = copy(wr_ref.at[pl.ds(kh + kq, kq)], wo_ref.at[pl.ds(kh + kq, kq)], 7, left)
        fl1.start()
        fl2.start()

        wrB.wait_recv()
        bl_ref[...] = (dot(0, k, wl_ref, 0, k) * scale).astype(jnp.bfloat16)
        sbl = copy(bl_ref, rbr_ref, 8, left)
        sbl.start()
        wlA.wait_recv()
        br_ref[...] = (dot(0, k, wr_ref, 0, k) * scale).astype(jnp.bfloat16)
        sbr = copy(br_ref, rbl_ref, 9, right)
        sbr.start()

        fr1.wait_recv()
        fl1.wait_recv()
        acc_ref[...] = dot(0, kq, wo_ref, 0, kq) + dot(kh, kq, wo_ref, kh, kq)
        fr2.wait_recv()
        fl2.wait_recv()
        mh = m_per // 2
        sbo = []
        for i, r0 in enumerate((0, mh)):
            bo_ref[pl.ds(r0, mh), :] = (
                (acc_ref[pl.ds(r0, mh), :]
                 + dot(kq, kq, wo_ref, kq, kq, r0, mh)
                 + dot(kh + kq, kq, wo_ref, kh + kq, kq, r0, mh)) * scale
            ).astype(jnp.bfloat16)
            d = copy(bo_ref.at[pl.ds(r0, mh)], rbo_ref.at[pl.ds(r0, mh)],
                     10 + i, opp)
            d.start()
            sbo.append(d)

        sbr.wait_recv()
        out_ref[pl.ds(left * m_per, m_per), :] = rbl_ref[...].astype(jnp.float32)
        sbl.wait_recv()
        out_ref[pl.ds(right * m_per, m_per), :] = rbr_ref[...].astype(jnp.float32)
        for i, r0 in enumerate((0, mh)):
            sbo[i].wait_recv()
            out_ref[pl.ds(opp * m_per + r0, mh), :] = (
                rbo_ref[pl.ds(r0, mh), :].astype(jnp.float32))

        for d in (wrA, wrB, wlB, wlA, fr1, fr2, fl1, fl2, sbl, sbr, *sbo):
            d.wait_send()

    return pl.pallas_call(
        body,
        out_shape=jax.ShapeDtypeStruct((m_tot, n_per), jnp.float32),
        in_specs=[
            pl.BlockSpec(memory_space=pltpu.VMEM),
            pl.BlockSpec(memory_space=pltpu.VMEM),
            pl.BlockSpec(memory_space=pltpu.SMEM),
            pl.BlockSpec(memory_space=pltpu.SMEM),
        ],
        out_specs=pl.BlockSpec(memory_space=pltpu.VMEM),
        scratch_shapes=[
            pltpu.VMEM((m_per, k), jnp.float8_e5m2),
            pltpu.VMEM((k, n_per), jnp.float8_e5m2),
            pltpu.VMEM((k, n_per), jnp.float8_e5m2),
            pltpu.VMEM((k, n_per), jnp.float8_e5m2),
            pltpu.VMEM((k, n_per), jnp.float8_e5m2),
            pltpu.VMEM((m_per, n_per), jnp.bfloat16),
            pltpu.VMEM((m_per, n_per), jnp.bfloat16),
            pltpu.VMEM((m_per, n_per), jnp.bfloat16),
            pltpu.VMEM((m_per, n_per), jnp.bfloat16),
            pltpu.VMEM((m_per, n_per), jnp.bfloat16),
            pltpu.VMEM((m_per, n_per), jnp.bfloat16),
            pltpu.VMEM((m_per, n_per), jnp.float32),
            pltpu.SemaphoreType.DMA((12,)),
            pltpu.SemaphoreType.DMA((12,)),
        ],
        compiler_params=pltpu.CompilerParams(
            collective_id=0,
            vmem_limit_bytes=100 * 1024 * 1024,
        ),
    )(x, w_mat, scale_x, scale_w)
